# baseline (speedup 1.0000x reference)
import functools

import jax
import jax.numpy as jnp
from jax import lax
from jax.experimental import pallas as pl
from jax.experimental.pallas import tpu as pltpu
from jax.experimental.pallas import tpu_sc as plsc

VOCAB = 100000
D_VE = 64
D_PAD = 128
KV_DIM = 1024
B = 4
S = 8192

NW = 32
N_TOK = B * S
TOK_PER_W = N_TOK // NW
CHUNK = 128
N_CHUNK = TOK_PER_W // CHUNK


HALF = N_CHUNK // 2


def _gather_kernel(table_hbm, ids_hbm, rows_hbm, idx_v, rows_v, sem):
    wid = lax.axis_index("s") * 2 + lax.axis_index("c")
    pltpu.sync_copy(ids_hbm.at[wid], idx_v)
    for h in range(2):
        copies = []
        for j in range(HALF):
            copies.append(
                pltpu.async_copy(
                    table_hbm.at[idx_v.at[h * HALF + j]], rows_v.at[j], sem
                )
            )
        for c in copies:
            c.wait()
        pltpu.sync_copy(rows_v, rows_hbm.at[wid, h])


def _sc_gather(table_pad, ids):
    mesh = plsc.VectorSubcoreMesh(core_axis_name="c", subcore_axis_name="s")
    k = functools.partial(
        pl.kernel,
        mesh=mesh,
        out_type=jax.ShapeDtypeStruct((NW, 2, HALF, CHUNK, D_PAD), jnp.float32),
        scratch_types=[
            pltpu.VMEM((N_CHUNK, CHUNK), jnp.int32),
            pltpu.VMEM((HALF, CHUNK, D_PAD), jnp.float32),
            pltpu.SemaphoreType.DMA,
        ],
        compiler_params=pltpu.CompilerParams(use_tc_tiling_on_sc=False),
    )(_gather_kernel)
    return k(table_pad, ids)


def _matmul_body(x_ref, w_ref, o_ref):
    o_ref[...] = jnp.dot(
        x_ref[...], w_ref[...], preferred_element_type=jnp.float32
    )


def _tc_project(rows, proj_pad):
    bm = 1024
    grid = (N_TOK // bm,)
    return pl.pallas_call(
        _matmul_body,
        grid=grid,
        in_specs=[
            pl.BlockSpec((bm, D_PAD), lambda i: (i, 0)),
            pl.BlockSpec((D_PAD, KV_DIM), lambda i: (0, 0)),
        ],
        out_specs=pl.BlockSpec((bm, KV_DIM), lambda i: (i, 0)),
        out_shape=jax.ShapeDtypeStruct((N_TOK, KV_DIM), jnp.float32),
    )(rows, proj_pad)


def kernel(input_ids, embed_weight, proj_weight):
    ids = input_ids.reshape(NW, N_CHUNK, CHUNK)
    table_pad = jnp.pad(embed_weight, ((0, 0), (0, D_PAD - D_VE)))
    proj_pad = jnp.pad(proj_weight, ((0, D_PAD - D_VE), (0, 0)))
    rows = _sc_gather(table_pad, ids).reshape(N_TOK, D_PAD)
    out = _tc_project(rows, proj_pad)
    return out.reshape(B, S, KV_DIM)


# speedup vs baseline: 1.0048x; 1.0048x over previous
import functools

import jax
import jax.numpy as jnp
from jax import lax
from jax.experimental import pallas as pl
from jax.experimental.pallas import tpu as pltpu
from jax.experimental.pallas import tpu_sc as plsc

VOCAB = 100000
D_VE = 64
D_PAD = 128
KV_DIM = 1024
B = 4
S = 8192

NW = 32
N_TOK = B * S
N_HALF = N_TOK // 2
TOK_PER_W = N_HALF // NW
CHUNK = 128
N_CHUNK = TOK_PER_W // CHUNK


def _gather_kernel(table_hbm, ids_hbm, rows_hbm, idx_v, rows_v, sem):
    wid = lax.axis_index("s") * 2 + lax.axis_index("c")
    pltpu.sync_copy(ids_hbm.at[wid], idx_v)
    copies = []
    for j in range(N_CHUNK):
        copies.append(
            pltpu.async_copy(table_hbm.at[idx_v.at[j]], rows_v.at[j], sem)
        )
    for c in copies:
        c.wait()
    pltpu.sync_copy(rows_v, rows_hbm.at[wid])


def _sc_gather(table_pad, ids):
    mesh = plsc.VectorSubcoreMesh(core_axis_name="c", subcore_axis_name="s")
    k = functools.partial(
        pl.kernel,
        mesh=mesh,
        out_type=jax.ShapeDtypeStruct((NW, N_CHUNK, CHUNK, D_PAD), jnp.float32),
        scratch_types=[
            pltpu.VMEM((N_CHUNK, CHUNK), jnp.int32),
            pltpu.VMEM((N_CHUNK, CHUNK, D_PAD), jnp.float32),
            pltpu.SemaphoreType.DMA,
        ],
        compiler_params=pltpu.CompilerParams(use_tc_tiling_on_sc=False),
    )(_gather_kernel)
    return k(table_pad, ids)


BM = 1024
NB_HALF = N_HALF // BM


def _matmul_body(x_ref, w_ref, o_ref):
    o_ref[...] = jnp.dot(
        x_ref[...], w_ref[...], preferred_element_type=jnp.float32
    )


def _matmul_body2(x_ref, w_ref, prev_ref, o_ref):
    del prev_ref
    o_ref[...] = jnp.dot(
        x_ref[...], w_ref[...], preferred_element_type=jnp.float32
    )


def _tc_project_first(rows, proj_pad):
    return pl.pallas_call(
        _matmul_body,
        grid=(NB_HALF,),
        in_specs=[
            pl.BlockSpec((BM, D_PAD), lambda i: (i, 0)),
            pl.BlockSpec((D_PAD, KV_DIM), lambda i: (0, 0)),
        ],
        out_specs=pl.BlockSpec((BM, KV_DIM), lambda i: (i, 0)),
        out_shape=jax.ShapeDtypeStruct((N_TOK, KV_DIM), jnp.float32),
    )(rows, proj_pad)


def _tc_project_second(rows, proj_pad, prev):
    return pl.pallas_call(
        _matmul_body2,
        grid=(NB_HALF,),
        in_specs=[
            pl.BlockSpec((BM, D_PAD), lambda i: (i, 0)),
            pl.BlockSpec((D_PAD, KV_DIM), lambda i: (0, 0)),
            pl.BlockSpec(memory_space=pl.ANY),
        ],
        out_specs=pl.BlockSpec((BM, KV_DIM), lambda i: (i + NB_HALF, 0)),
        out_shape=jax.ShapeDtypeStruct((N_TOK, KV_DIM), jnp.float32),
        input_output_aliases={2: 0},
    )(rows, proj_pad, prev)


def kernel(input_ids, embed_weight, proj_weight):
    flat = input_ids.reshape(2, NW, N_CHUNK, CHUNK)
    table_pad = jnp.pad(embed_weight, ((0, 0), (0, D_PAD - D_VE)))
    proj_pad = jnp.pad(proj_weight, ((0, D_PAD - D_VE), (0, 0)))
    rows1 = _sc_gather(table_pad, flat[0]).reshape(N_HALF, D_PAD)
    rows2 = _sc_gather(table_pad, flat[1]).reshape(N_HALF, D_PAD)
    out = _tc_project_first(rows1, proj_pad)
    out = _tc_project_second(rows2, proj_pad, out)
    return out.reshape(B, S, KV_DIM)
